# baseline (speedup 1.0000x reference)
import functools

import jax
import jax.numpy as jnp
import numpy as np
from jax import lax
from jax.experimental import pallas as pl
from jax.experimental.pallas import tpu as pltpu
from jax.experimental.pallas import tpu_sc as plsc

N = 10000
E = 320000
D = 128
B = 8

NC = 2
NS = 16
NW = NC * NS
EW = E // NW
CH = 128
NFULL = EW // CH
TAIL = EW - NFULL * CH
TRB = 624
TEX = N - NS * TRB
ZR = 104

RB = 1000
NRB = N // RB
RB1 = 2000
NRB1 = N // RB1
EB = 512
NEB = E // EB


def _iota_oh(b):
    return (b[:, None] == lax.broadcasted_iota(jnp.int32, (1, B), 1)).astype(
        jnp.float32)


def _pre_body(x_ref, b3_ref, u_ref, wa_ref, wb_ref, wd_ref, be1_ref,
              p1_ref, p2_ref):
    x = x_ref[...]
    oh = _iota_oh(b3_ref[0, 0, :])
    u1 = jnp.dot(u_ref[...], wd_ref[...]) + be1_ref[...]
    p1_ref[...] = (jnp.dot(x, wa_ref[...])
                   + jnp.dot(oh, u1)).astype(jnp.bfloat16)
    p2_ref[...] = jnp.dot(x, wb_ref[...]).astype(jnp.bfloat16)


def _precompute(x, batch3, u, wa, wb, wd, be1):
    return pl.pallas_call(
        _pre_body,
        grid=(NRB1,),
        in_specs=[
            pl.BlockSpec((RB1, D), lambda i: (i, 0)),
            pl.BlockSpec((1, 1, RB1), lambda i: (i, 0, 0)),
            pl.BlockSpec((B, D), lambda i: (0, 0)),
            pl.BlockSpec((D, D), lambda i: (0, 0)),
            pl.BlockSpec((D, D), lambda i: (0, 0)),
            pl.BlockSpec((D, D), lambda i: (0, 0)),
            pl.BlockSpec((D,), lambda i: (0,)),
        ],
        out_specs=[
            pl.BlockSpec((RB1, D), lambda i: (i, 0)),
            pl.BlockSpec((RB1, D), lambda i: (i, 0)),
        ],
        out_shape=[
            jax.ShapeDtypeStruct((N, D), jnp.bfloat16),
            jax.ShapeDtypeStruct((N, D), jnp.bfloat16),
        ],
    )(x, batch3, u, wa, wb, wd, be1)


def _sc_gather(p1, p2, src, dst, batchv):
    mesh = plsc.VectorSubcoreMesh(core_axis_name="c", subcore_axis_name="s",
                                  num_cores=NC, num_subcores=NS)

    @functools.partial(
        pl.kernel,
        out_type=(
            jax.ShapeDtypeStruct((E, D // 2), jnp.int32),
            jax.ShapeDtypeStruct((E, D // 2), jnp.int32),
            jax.ShapeDtypeStruct((E,), jnp.int32),
        ),
        mesh=mesh,
        compiler_params=pltpu.CompilerParams(needs_layout_passes=False,
                                             use_tc_tiling_on_sc=False),
        scratch_types=[
            pltpu.VMEM((EW,), jnp.int32),
            pltpu.VMEM((EW,), jnp.int32),
            pltpu.VMEM((EW,), jnp.int32),
            pltpu.VMEM((N,), jnp.int32),
            pltpu.VMEM((2, CH, D // 2), jnp.int32),
            pltpu.VMEM((2, CH, D // 2), jnp.int32),
            pltpu.VMEM((TAIL, D // 2), jnp.int32),
            pltpu.VMEM((TAIL, D // 2), jnp.int32),
            pltpu.SemaphoreType.DMA,
            pltpu.SemaphoreType.DMA,
        ],
    )
    def k(p1_hbm, p2_hbm, src_hbm, dst_hbm, batch_hbm,
          gs_hbm, gd_hbm, bsrc_hbm,
          sidx_all, didx_all, bsrc_all, batch_v,
          rows_a, rows_b, rows_at, rows_bt, gsem, wsem):
        wid = lax.axis_index("s") * NC + lax.axis_index("c")
        wbase = wid * EW
        pltpu.sync_copy(src_hbm.at[pl.ds(wbase, EW)], sidx_all)
        pltpu.sync_copy(dst_hbm.at[pl.ds(wbase, EW)], didx_all)
        pltpu.sync_copy(batch_hbm, batch_v)

        def fire(j, buf):
            pltpu.async_copy(
                p1_hbm.at[sidx_all.at[pl.ds(j * CH, CH)]], rows_a.at[buf],
                gsem)
            pltpu.async_copy(
                p2_hbm.at[didx_all.at[pl.ds(j * CH, CH)]], rows_b.at[buf],
                gsem)

        fire(0, 0)

        def body(j, carry):
            p = j & 1
            q = 1 - p
            pltpu.make_async_copy(p1_hbm.at[sidx_all.at[pl.ds(0, CH)]],
                                  rows_a.at[p], gsem).wait()
            pltpu.make_async_copy(p1_hbm.at[sidx_all.at[pl.ds(0, CH)]],
                                  rows_b.at[p], gsem).wait()

            @pl.when(j >= 1)
            def _():
                pltpu.make_async_copy(rows_a.at[q],
                                      gs_hbm.at[pl.ds(0, CH)], wsem).wait()
                pltpu.make_async_copy(rows_b.at[q],
                                      gd_hbm.at[pl.ds(0, CH)], wsem).wait()

            @pl.when(j < NFULL - 1)
            def _():
                fire(j + 1, q)

            for kk in range(CH // 16):
                off = pl.ds(j * CH + kk * 16, 16)
                bsrc_all[off] = plsc.load_gather(batch_v, [sidx_all[off]])
            pltpu.async_copy(rows_a.at[p],
                             gs_hbm.at[pl.ds(wbase + j * CH, CH)], wsem)
            pltpu.async_copy(rows_b.at[p],
                             gd_hbm.at[pl.ds(wbase + j * CH, CH)], wsem)
            return carry

        lax.fori_loop(0, NFULL, body, 0)
        pltpu.make_async_copy(rows_a.at[(NFULL - 1) & 1],
                              gs_hbm.at[pl.ds(0, CH)], wsem).wait()
        pltpu.make_async_copy(rows_b.at[(NFULL - 1) & 1],
                              gd_hbm.at[pl.ds(0, CH)], wsem).wait()

        tb = NFULL * CH
        c1 = pltpu.async_copy(p1_hbm.at[sidx_all.at[pl.ds(tb, TAIL)]],
                              rows_at, gsem)
        c2 = pltpu.async_copy(p2_hbm.at[didx_all.at[pl.ds(tb, TAIL)]],
                              rows_bt, gsem)
        c1.wait()
        c2.wait()
        off = pl.ds(tb, TAIL)
        bsrc_all[off] = plsc.load_gather(batch_v, [sidx_all[off]])
        pltpu.sync_copy(rows_at, gs_hbm.at[pl.ds(wbase + tb, TAIL)])
        pltpu.sync_copy(rows_bt, gd_hbm.at[pl.ds(wbase + tb, TAIL)])
        pltpu.sync_copy(bsrc_all, bsrc_hbm.at[pl.ds(wbase, EW)])

    return k(p1, p2, src, dst, batchv)


def _unpack_bf16_pair(w):
    lo = lax.bitcast_convert_type(w << 16, jnp.float32)
    hi = lax.bitcast_convert_type(w & jnp.int32(-65536), jnp.float32)
    return jnp.concatenate([lo, hi], axis=1)


def _edge_body(ea_ref, gs_ref, gd_ref, b3_ref, wc_ref, w2_ref, be2_ref,
               en_ref, eg_ref, acc_ref):
    i = pl.program_id(0)
    ea = ea_ref[...]
    g = _unpack_bf16_pair(gs_ref[...]) + _unpack_bf16_pair(gd_ref[...])
    h = jnp.maximum(g + jnp.dot(ea, wc_ref[...]), 0.0)
    en = ea + jnp.dot(h, w2_ref[...]) + be2_ref[...]
    en_ref[...] = en
    oh = _iota_oh(b3_ref[0, 0, :])

    @pl.when(i == 0)
    def _():
        acc_ref[...] = jnp.zeros_like(acc_ref)

    acc_ref[...] += lax.dot_general(oh, en, (((0,), (0,)), ((), ())))

    @pl.when(i == NEB - 1)
    def _():
        eg_ref[...] = acc_ref[...]


def _edge_mlp(edge_attr, gs, gd, bsrc3, wc, w2, be2):
    return pl.pallas_call(
        _edge_body,
        grid=(NEB,),
        in_specs=[
            pl.BlockSpec((EB, D), lambda i: (i, 0)),
            pl.BlockSpec((EB, D // 2), lambda i: (i, 0)),
            pl.BlockSpec((EB, D // 2), lambda i: (i, 0)),
            pl.BlockSpec((1, 1, EB), lambda i: (i, 0, 0)),
            pl.BlockSpec((D, D), lambda i: (0, 0)),
            pl.BlockSpec((D, D), lambda i: (0, 0)),
            pl.BlockSpec((D,), lambda i: (0,)),
        ],
        out_specs=[
            pl.BlockSpec((EB, D), lambda i: (i, 0)),
            pl.BlockSpec((B, D), lambda i: (0, 0)),
        ],
        out_shape=[
            jax.ShapeDtypeStruct((E, D), jnp.float32),
            jax.ShapeDtypeStruct((B, D), jnp.float32),
        ],
        scratch_shapes=[pltpu.VMEM((B, D), jnp.float32)],
    )(edge_attr, gs, gd, bsrc3, wc, w2, be2)


def _sc_scatter(e_new, dst):
    mesh = plsc.VectorSubcoreMesh(core_axis_name="c", subcore_axis_name="s",
                                  num_cores=NC, num_subcores=NS)

    @functools.partial(
        pl.kernel,
        out_type=jax.ShapeDtypeStruct((NC * N, D), jnp.float32),
        mesh=mesh,
        compiler_params=pltpu.CompilerParams(needs_layout_passes=False),
        scratch_types=[
            pltpu.VMEM_SHARED((N, D), jnp.float32),
            pltpu.VMEM((2, CH), jnp.int32),
            pltpu.VMEM((2, CH, D), jnp.float32),
            pltpu.VMEM((TAIL,), jnp.int32),
            pltpu.VMEM((TAIL, D), jnp.float32),
            pltpu.VMEM((ZR, D), jnp.float32),
            pltpu.SemaphoreType.DMA,
            pltpu.SemaphoreType.DMA,
        ],
    )
    def k(en_hbm, dst_hbm, out_hbm,
          spmem, didx, rows, didx_t, rows_t, zbuf, isem, rsem):
        cid = lax.axis_index("c")
        sid = lax.axis_index("s")
        wid = sid * NC + cid

        def zb(r, carry):
            for kk in range(D // 16):
                zbuf[r, pl.ds(kk * 16, 16)] = jnp.zeros((16,), jnp.float32)
            return carry

        lax.fori_loop(0, ZR, zb, 0)
        tbase = sid * TRB
        for kk in range(TRB // ZR):
            pltpu.sync_copy(zbuf, spmem.at[pl.ds(tbase + kk * ZR, ZR)])

        @pl.when(sid == 0)
        def _():
            pltpu.sync_copy(zbuf.at[pl.ds(0, TEX)],
                            spmem.at[pl.ds(NS * TRB, TEX)])

        plsc.subcore_barrier()

        def fire(j, buf):
            base = wid * EW + j * CH
            pltpu.async_copy(dst_hbm.at[pl.ds(base, CH)], didx.at[buf], isem)
            pltpu.async_copy(en_hbm.at[pl.ds(base, CH)], rows.at[buf], rsem)

        fire(0, 0)

        def body(j, carry):
            p = j & 1
            q = 1 - p
            pltpu.make_async_copy(dst_hbm.at[pl.ds(0, CH)],
                                  didx.at[p], isem).wait()
            pltpu.make_async_copy(en_hbm.at[pl.ds(0, CH)],
                                  rows.at[p], rsem).wait()

            @pl.when(j < NFULL - 1)
            def _():
                fire(j + 1, q)

            pltpu.sync_copy(rows.at[p], spmem.at[didx.at[p]], add=True)
            return carry

        lax.fori_loop(0, NFULL, body, 0)
        tb = wid * EW + NFULL * CH
        pltpu.sync_copy(dst_hbm.at[pl.ds(tb, TAIL)], didx_t)
        pltpu.sync_copy(en_hbm.at[pl.ds(tb, TAIL)], rows_t)
        pltpu.sync_copy(rows_t, spmem.at[didx_t], add=True)
        plsc.subcore_barrier()
        pltpu.sync_copy(spmem.at[pl.ds(tbase, TRB)],
                        out_hbm.at[pl.ds(cid * N + tbase, TRB)])

        @pl.when(sid == 0)
        def _():
            pltpu.sync_copy(spmem.at[pl.ds(NS * TRB, TEX)],
                            out_hbm.at[pl.ds(cid * N + NS * TRB, TEX)])

    return k(e_new, dst)


def _node_body(x_ref, aa_ref, ab_ref, b3_ref, u_ref, eg_ref,
               wna_ref, wnb_ref, wnc_ref, bn1_ref, wn2_ref, bn2_ref,
               wga_ref, wgb_ref, wgc_ref, bg1_ref, wg2_ref, bg2_ref,
               xn_ref, un_ref, acc_ref):
    i = pl.program_id(0)
    x = x_ref[...]
    agg = aa_ref[...] + ab_ref[...]
    oh = _iota_oh(b3_ref[0, 0, :])
    u = u_ref[...]
    u3 = jnp.dot(u, wnc_ref[...])
    h = jnp.maximum(
        jnp.dot(x, wna_ref[...]) + jnp.dot(agg, wnb_ref[...])
        + jnp.dot(oh, u3) + bn1_ref[...], 0.0)
    xn = x + jnp.dot(h, wn2_ref[...]) + bn2_ref[...]
    xn_ref[...] = xn

    @pl.when(i == 0)
    def _():
        acc_ref[...] = jnp.zeros_like(acc_ref)

    acc_ref[...] += lax.dot_general(oh, xn, (((0,), (0,)), ((), ())))

    @pl.when(i == NRB - 1)
    def _():
        ng = acc_ref[...]
        g = jnp.maximum(
            jnp.dot(ng, wga_ref[...]) + jnp.dot(eg_ref[...], wgb_ref[...])
            + jnp.dot(u, wgc_ref[...]) + bg1_ref[...], 0.0)
        un_ref[...] = u + jnp.dot(g, wg2_ref[...]) + bg2_ref[...]


def _node_global(x, aggp, batch3, u, eg,
                 wna, wnb, wnc, bn1, wn2, bn2,
                 wga, wgb, wgc, bg1, wg2, bg2):
    wspec = pl.BlockSpec((D, D), lambda i: (0, 0))
    bspec = pl.BlockSpec((D,), lambda i: (0,))
    return pl.pallas_call(
        _node_body,
        grid=(NRB,),
        in_specs=[
            pl.BlockSpec((RB, D), lambda i: (i, 0)),
            pl.BlockSpec((RB, D), lambda i: (i, 0)),
            pl.BlockSpec((RB, D), lambda i: (i + NRB, 0)),
            pl.BlockSpec((1, 1, RB), lambda i: (i, 0, 0)),
            pl.BlockSpec((B, D), lambda i: (0, 0)),
            pl.BlockSpec((B, D), lambda i: (0, 0)),
            wspec, wspec, wspec, bspec, wspec, bspec,
            wspec, wspec, wspec, bspec, wspec, bspec,
        ],
        out_specs=[
            pl.BlockSpec((RB, D), lambda i: (i, 0)),
            pl.BlockSpec((B, D), lambda i: (0, 0)),
        ],
        out_shape=[
            jax.ShapeDtypeStruct((N, D), jnp.float32),
            jax.ShapeDtypeStruct((B, D), jnp.float32),
        ],
        scratch_shapes=[pltpu.VMEM((B, D), jnp.float32)],
    )(x, aggp, aggp, batch3, u, eg,
      wna, wnb, wnc, bn1, wn2, bn2,
      wga, wgb, wgc, bg1, wg2, bg2)


def kernel(x, edge_attr, u, edge_index, batch,
           We1, be1, We2, be2,
           Wn1, bn1, Wn2, bn2,
           Wg1, bg1, Wg2, bg2):
    src = edge_index[0].astype(jnp.int32)
    dst = edge_index[1].astype(jnp.int32)
    batch32 = batch.astype(jnp.int32)
    batch3 = batch32.reshape(NRB, 1, RB)

    p1, p2 = _precompute(x, batch32.reshape(NRB1, 1, RB1), u,
                         We1[:D], We1[D:2 * D], We1[3 * D:], be1)
    p1i = lax.bitcast_convert_type(p1.reshape(N, D // 2, 2), jnp.int32)
    p2i = lax.bitcast_convert_type(p2.reshape(N, D // 2, 2), jnp.int32)
    gs, gd, bsrc = _sc_gather(p1i, p2i, src, dst, batch32)
    perm = np.concatenate([np.arange(0, D, 2), np.arange(1, D, 2)])
    e_new, edge_g = _edge_mlp(edge_attr, gs, gd,
                              bsrc.reshape(NEB, 1, EB),
                              We1[2 * D:3 * D][:, perm], We2[perm, :], be2)
    aggp = _sc_scatter(e_new, dst)
    x_new, u_new = _node_global(
        x, aggp, batch3, u, edge_g,
        Wn1[:D], Wn1[D:2 * D], Wn1[2 * D:], bn1, Wn2, bn2,
        Wg1[:D], Wg1[D:2 * D], Wg1[2 * D:], bg1, Wg2, bg2)
    return (x_new, e_new, u_new)


# speedup vs baseline: 1.1780x; 1.1780x over previous
import functools

import jax
import jax.numpy as jnp
import numpy as np
from jax import lax
from jax.experimental import pallas as pl
from jax.experimental.pallas import tpu as pltpu
from jax.experimental.pallas import tpu_sc as plsc

N = 10000
E = 320000
D = 128
B = 8

NC = 2
NS = 16
NW = NC * NS
CH = 128

EB = 512
NEB0 = 312
E0 = NEB0 * EB
E1 = E - E0
NEB1 = E1 // EB

TRB = 624
TEX = N - NS * TRB
ZR = 104

RB = 1000
NRB = N // RB
RB1 = 2000
NRB1 = N // RB1


def _iota_oh(b):
    return (b[:, None] == lax.broadcasted_iota(jnp.int32, (1, B), 1)).astype(
        jnp.float32)


def _pre_body(x_ref, b3_ref, u_ref, wa_ref, wb_ref, wd_ref, be1_ref,
              p1_ref, p2_ref):
    x = x_ref[...]
    oh = _iota_oh(b3_ref[0, 0, :])
    u1 = jnp.dot(u_ref[...], wd_ref[...]) + be1_ref[...]
    p1_ref[...] = jnp.dot(x, wa_ref[...]) + jnp.dot(oh, u1)
    p2_ref[...] = jnp.dot(x, wb_ref[...])


def _precompute(x, batch3, u, wa, wb, wd, be1):
    return pl.pallas_call(
        _pre_body,
        grid=(NRB1,),
        in_specs=[
            pl.BlockSpec((RB1, D), lambda i: (i, 0)),
            pl.BlockSpec((1, 1, RB1), lambda i: (i, 0, 0)),
            pl.BlockSpec((B, D), lambda i: (0, 0)),
            pl.BlockSpec((D, D), lambda i: (0, 0)),
            pl.BlockSpec((D, D), lambda i: (0, 0)),
            pl.BlockSpec((D, D), lambda i: (0, 0)),
            pl.BlockSpec((D,), lambda i: (0,)),
        ],
        out_specs=[
            pl.BlockSpec((RB1, D), lambda i: (i, 0)),
            pl.BlockSpec((RB1, D), lambda i: (i, 0)),
        ],
        out_shape=[
            jax.ShapeDtypeStruct((N, D), jnp.float32),
            jax.ShapeDtypeStruct((N, D), jnp.float32),
        ],
    )(x, batch3, u, wa, wb, wd, be1)


def _sc_gather(p1, p2, srch, dsth, batchv, eh):
    ew = eh // NW
    nfull = ew // CH
    tail = ew - nfull * CH
    mesh = plsc.VectorSubcoreMesh(core_axis_name="c", subcore_axis_name="s",
                                  num_cores=NC, num_subcores=NS)

    @functools.partial(
        pl.kernel,
        out_type=(
            jax.ShapeDtypeStruct((eh, D), jnp.float32),
            jax.ShapeDtypeStruct((eh, D), jnp.float32),
            jax.ShapeDtypeStruct((eh,), jnp.int32),
        ),
        mesh=mesh,
        compiler_params=pltpu.CompilerParams(needs_layout_passes=False),
        scratch_types=[
            pltpu.VMEM((ew,), jnp.int32),
            pltpu.VMEM((ew,), jnp.int32),
            pltpu.VMEM((ew,), jnp.int32),
            pltpu.VMEM((N,), jnp.int32),
            pltpu.VMEM((2, CH, D), jnp.float32),
            pltpu.VMEM((2, CH, D), jnp.float32),
            pltpu.VMEM((max(tail, 8), D), jnp.float32),
            pltpu.VMEM((max(tail, 8), D), jnp.float32),
            pltpu.SemaphoreType.DMA,
            pltpu.SemaphoreType.DMA,
        ],
    )
    def k(p1_hbm, p2_hbm, src_hbm, dst_hbm, batch_hbm,
          gs_hbm, gd_hbm, bsrc_hbm,
          sidx_all, didx_all, bsrc_all, batch_v,
          rows_a, rows_b, rows_at, rows_bt, gsem, wsem):
        wid = lax.axis_index("s") * NC + lax.axis_index("c")
        wbase = wid * ew
        pltpu.sync_copy(src_hbm.at[pl.ds(wbase, ew)], sidx_all)
        pltpu.sync_copy(dst_hbm.at[pl.ds(wbase, ew)], didx_all)
        pltpu.sync_copy(batch_hbm, batch_v)

        def fire(j, buf):
            pltpu.async_copy(
                p1_hbm.at[sidx_all.at[pl.ds(j * CH, CH)]], rows_a.at[buf],
                gsem)
            pltpu.async_copy(
                p2_hbm.at[didx_all.at[pl.ds(j * CH, CH)]], rows_b.at[buf],
                gsem)

        fire(0, 0)

        def body(j, carry):
            p = j & 1
            q = 1 - p
            pltpu.make_async_copy(p1_hbm.at[sidx_all.at[pl.ds(0, CH)]],
                                  rows_a.at[p], gsem).wait()
            pltpu.make_async_copy(p1_hbm.at[sidx_all.at[pl.ds(0, CH)]],
                                  rows_b.at[p], gsem).wait()

            @pl.when(j >= 1)
            def _():
                pltpu.make_async_copy(rows_a.at[q],
                                      gs_hbm.at[pl.ds(0, CH)], wsem).wait()
                pltpu.make_async_copy(rows_b.at[q],
                                      gd_hbm.at[pl.ds(0, CH)], wsem).wait()

            @pl.when(j < nfull - 1)
            def _():
                fire(j + 1, q)

            for kk in range(CH // 16):
                off = pl.ds(j * CH + kk * 16, 16)
                bsrc_all[off] = plsc.load_gather(batch_v, [sidx_all[off]])
            pltpu.async_copy(rows_a.at[p],
                             gs_hbm.at[pl.ds(wbase + j * CH, CH)], wsem)
            pltpu.async_copy(rows_b.at[p],
                             gd_hbm.at[pl.ds(wbase + j * CH, CH)], wsem)
            return carry

        lax.fori_loop(0, nfull, body, 0)
        pltpu.make_async_copy(rows_a.at[(nfull - 1) & 1],
                              gs_hbm.at[pl.ds(0, CH)], wsem).wait()
        pltpu.make_async_copy(rows_b.at[(nfull - 1) & 1],
                              gd_hbm.at[pl.ds(0, CH)], wsem).wait()

        if tail:
            tb = nfull * CH
            c1 = pltpu.async_copy(p1_hbm.at[sidx_all.at[pl.ds(tb, tail)]],
                                  rows_at, gsem)
            c2 = pltpu.async_copy(p2_hbm.at[didx_all.at[pl.ds(tb, tail)]],
                                  rows_bt, gsem)
            c1.wait()
            c2.wait()
            off = pl.ds(tb, tail)
            bsrc_all[off] = plsc.load_gather(batch_v, [sidx_all[off]])
            pltpu.sync_copy(rows_at, gs_hbm.at[pl.ds(wbase + tb, tail)])
            pltpu.sync_copy(rows_bt, gd_hbm.at[pl.ds(wbase + tb, tail)])
        pltpu.sync_copy(bsrc_all, bsrc_hbm.at[pl.ds(wbase, ew)])

    return k(p1, p2, srch, dsth, batchv)


def _edge_body(nblk, ea_ref, gs_ref, gd_ref, b3_ref, wc_ref, w2_ref, be2_ref,
               en_ref, eg_ref, acc_ref):
    i = pl.program_id(0)
    ea = ea_ref[...]
    h = jnp.maximum(gs_ref[...] + gd_ref[...] + jnp.dot(ea, wc_ref[...]), 0.0)
    en = ea + jnp.dot(h, w2_ref[...]) + be2_ref[...]
    en_ref[...] = en
    oh = _iota_oh(b3_ref[0, 0, :])

    @pl.when(i == 0)
    def _():
        acc_ref[...] = jnp.zeros_like(acc_ref)

    acc_ref[...] += lax.dot_general(oh, en, (((0,), (0,)), ((), ())))

    @pl.when(i == nblk - 1)
    def _():
        eg_ref[...] = acc_ref[...]


def _edge_mlp_half(edge_attr, gs, gd, bsrc3, wc, w2, be2, off_b, nblk,
                   en_prev=None):
    ebs = pl.BlockSpec((EB, D), lambda i: (i + off_b, 0))
    kwargs = {}
    ins = [edge_attr, gs, gd, bsrc3, wc, w2, be2]
    in_specs = [
        ebs,
        pl.BlockSpec((EB, D), lambda i: (i, 0)),
        pl.BlockSpec((EB, D), lambda i: (i, 0)),
        pl.BlockSpec((1, 1, EB), lambda i: (i, 0, 0)),
        pl.BlockSpec((D, D), lambda i: (0, 0)),
        pl.BlockSpec((D, D), lambda i: (0, 0)),
        pl.BlockSpec((D,), lambda i: (0,)),
    ]
    if en_prev is not None:
        ins.append(en_prev)
        in_specs.append(pl.BlockSpec(memory_space=pl.ANY))
        kwargs["input_output_aliases"] = {7: 0}

    def body(*refs):
        if en_prev is not None:
            refs = refs[:7] + refs[8:]
        _edge_body(nblk, *refs)

    return pl.pallas_call(
        body,
        grid=(nblk,),
        in_specs=in_specs,
        out_specs=[
            ebs,
            pl.BlockSpec((B, D), lambda i: (0, 0)),
        ],
        out_shape=[
            jax.ShapeDtypeStruct((E, D), jnp.float32),
            jax.ShapeDtypeStruct((B, D), jnp.float32),
        ],
        scratch_shapes=[pltpu.VMEM((B, D), jnp.float32)],
        **kwargs,
    )(*ins)


def _sc_scatter(e_new, dsth, eoff, eh):
    ew = eh // NW
    nfull = ew // CH
    tail = ew - nfull * CH
    mesh = plsc.VectorSubcoreMesh(core_axis_name="c", subcore_axis_name="s",
                                  num_cores=NC, num_subcores=NS)

    @functools.partial(
        pl.kernel,
        out_type=jax.ShapeDtypeStruct((NC * N, D), jnp.float32),
        mesh=mesh,
        compiler_params=pltpu.CompilerParams(needs_layout_passes=False),
        scratch_types=[
            pltpu.VMEM_SHARED((N, D), jnp.float32),
            pltpu.VMEM((2, CH), jnp.int32),
            pltpu.VMEM((2, CH, D), jnp.float32),
            pltpu.VMEM((max(tail, 8),), jnp.int32),
            pltpu.VMEM((max(tail, 8), D), jnp.float32),
            pltpu.VMEM((ZR, D), jnp.float32),
            pltpu.SemaphoreType.DMA,
            pltpu.SemaphoreType.DMA,
        ],
    )
    def k(en_hbm, dst_hbm, out_hbm,
          spmem, didx, rows, didx_t, rows_t, zbuf, isem, rsem):
        cid = lax.axis_index("c")
        sid = lax.axis_index("s")
        wid = sid * NC + cid

        def zb(r, carry):
            for kk in range(D // 16):
                zbuf[r, pl.ds(kk * 16, 16)] = jnp.zeros((16,), jnp.float32)
            return carry

        lax.fori_loop(0, ZR, zb, 0)
        tbase = sid * TRB
        for kk in range(TRB // ZR):
            pltpu.sync_copy(zbuf, spmem.at[pl.ds(tbase + kk * ZR, ZR)])

        @pl.when(sid == 0)
        def _():
            pltpu.sync_copy(zbuf.at[pl.ds(0, TEX)],
                            spmem.at[pl.ds(NS * TRB, TEX)])

        plsc.subcore_barrier()

        def fire(j, buf):
            base = wid * ew + j * CH
            pltpu.async_copy(dst_hbm.at[pl.ds(base, CH)], didx.at[buf], isem)
            pltpu.async_copy(en_hbm.at[pl.ds(eoff + base, CH)],
                             rows.at[buf], rsem)

        fire(0, 0)

        def body(j, carry):
            p = j & 1
            q = 1 - p
            pltpu.make_async_copy(dst_hbm.at[pl.ds(0, CH)],
                                  didx.at[p], isem).wait()
            pltpu.make_async_copy(en_hbm.at[pl.ds(0, CH)],
                                  rows.at[p], rsem).wait()

            @pl.when(j < nfull - 1)
            def _():
                fire(j + 1, q)

            pltpu.sync_copy(rows.at[p], spmem.at[didx.at[p]], add=True)
            return carry

        lax.fori_loop(0, nfull, body, 0)
        if tail:
            tb = wid * ew + nfull * CH
            pltpu.sync_copy(dst_hbm.at[pl.ds(tb, tail)], didx_t)
            pltpu.sync_copy(en_hbm.at[pl.ds(eoff + tb, tail)], rows_t)
            pltpu.sync_copy(rows_t, spmem.at[didx_t], add=True)
        plsc.subcore_barrier()
        pltpu.sync_copy(spmem.at[pl.ds(tbase, TRB)],
                        out_hbm.at[pl.ds(cid * N + tbase, TRB)])

        @pl.when(sid == 0)
        def _():
            pltpu.sync_copy(spmem.at[pl.ds(NS * TRB, TEX)],
                            out_hbm.at[pl.ds(cid * N + NS * TRB, TEX)])

    return k(e_new, dsth)


def _node_body(x_ref, a0_ref, a1_ref, a2_ref, a3_ref, b3_ref, u_ref,
               eg0_ref, eg1_ref,
               wna_ref, wnb_ref, wnc_ref, bn1_ref, wn2_ref, bn2_ref,
               wga_ref, wgb_ref, wgc_ref, bg1_ref, wg2_ref, bg2_ref,
               xn_ref, un_ref, acc_ref):
    i = pl.program_id(0)
    x = x_ref[...]
    agg = (a0_ref[...] + a1_ref[...]) + (a2_ref[...] + a3_ref[...])
    oh = _iota_oh(b3_ref[0, 0, :])
    u = u_ref[...]
    u3 = jnp.dot(u, wnc_ref[...])
    h = jnp.maximum(
        jnp.dot(x, wna_ref[...]) + jnp.dot(agg, wnb_ref[...])
        + jnp.dot(oh, u3) + bn1_ref[...], 0.0)
    xn = x + jnp.dot(h, wn2_ref[...]) + bn2_ref[...]
    xn_ref[...] = xn

    @pl.when(i == 0)
    def _():
        acc_ref[...] = jnp.zeros_like(acc_ref)

    acc_ref[...] += lax.dot_general(oh, xn, (((0,), (0,)), ((), ())))

    @pl.when(i == NRB - 1)
    def _():
        ng = acc_ref[...]
        eg = eg0_ref[...] + eg1_ref[...]
        g = jnp.maximum(
            jnp.dot(ng, wga_ref[...]) + jnp.dot(eg, wgb_ref[...])
            + jnp.dot(u, wgc_ref[...]) + bg1_ref[...], 0.0)
        un_ref[...] = u + jnp.dot(g, wg2_ref[...]) + bg2_ref[...]


def _node_global(x, aggp0, aggp1, batch3, u, eg0, eg1,
                 wna, wnb, wnc, bn1, wn2, bn2,
                 wga, wgb, wgc, bg1, wg2, bg2):
    wspec = pl.BlockSpec((D, D), lambda i: (0, 0))
    bspec = pl.BlockSpec((D,), lambda i: (0,))
    uspec = pl.BlockSpec((B, D), lambda i: (0, 0))
    return pl.pallas_call(
        _node_body,
        grid=(NRB,),
        in_specs=[
            pl.BlockSpec((RB, D), lambda i: (i, 0)),
            pl.BlockSpec((RB, D), lambda i: (i, 0)),
            pl.BlockSpec((RB, D), lambda i: (i + NRB, 0)),
            pl.BlockSpec((RB, D), lambda i: (i, 0)),
            pl.BlockSpec((RB, D), lambda i: (i + NRB, 0)),
            pl.BlockSpec((1, 1, RB), lambda i: (i, 0, 0)),
            uspec, uspec, uspec,
            wspec, wspec, wspec, bspec, wspec, bspec,
            wspec, wspec, wspec, bspec, wspec, bspec,
        ],
        out_specs=[
            pl.BlockSpec((RB, D), lambda i: (i, 0)),
            pl.BlockSpec((B, D), lambda i: (0, 0)),
        ],
        out_shape=[
            jax.ShapeDtypeStruct((N, D), jnp.float32),
            jax.ShapeDtypeStruct((B, D), jnp.float32),
        ],
        scratch_shapes=[pltpu.VMEM((B, D), jnp.float32)],
    )(x, aggp0, aggp0, aggp1, aggp1, batch3, u, eg0, eg1,
      wna, wnb, wnc, bn1, wn2, bn2,
      wga, wgb, wgc, bg1, wg2, bg2)


def kernel(x, edge_attr, u, edge_index, batch,
           We1, be1, We2, be2,
           Wn1, bn1, Wn2, bn2,
           Wg1, bg1, Wg2, bg2):
    src = edge_index[0].astype(jnp.int32)
    dst = edge_index[1].astype(jnp.int32)
    batch32 = batch.astype(jnp.int32)
    batch3 = batch32.reshape(NRB, 1, RB)

    p1, p2 = _precompute(x, batch32.reshape(NRB1, 1, RB1), u,
                         We1[:D], We1[D:2 * D], We1[3 * D:], be1)

    src0, src1 = src[:E0], src[E0:]
    dst0, dst1 = dst[:E0], dst[E0:]
    gs0, gd0, bsrc0 = _sc_gather(p1, p2, src0, dst0, batch32, E0)
    gs1, gd1, bsrc1 = _sc_gather(p1, p2, src1, dst1, batch32, E1)

    wc, w2 = We1[2 * D:3 * D], We2
    en0, eg0 = _edge_mlp_half(edge_attr, gs0, gd0,
                              bsrc0.reshape(NEB0, 1, EB), wc, w2, be2,
                              0, NEB0)
    e_new, eg1 = _edge_mlp_half(edge_attr, gs1, gd1,
                                bsrc1.reshape(NEB1, 1, EB), wc, w2, be2,
                                NEB0, NEB1, en_prev=en0)

    aggp0 = _sc_scatter(e_new, dst0, 0, E0)
    aggp1 = _sc_scatter(e_new, dst1, E0, E1)

    x_new, u_new = _node_global(
        x, aggp0, aggp1, batch3, u, eg0, eg1,
        Wn1[:D], Wn1[D:2 * D], Wn1[2 * D:], bn1, Wn2, bn2,
        Wg1[:D], Wg1[D:2 * D], Wg1[2 * D:], bg1, Wg2, bg2)
    return (x_new, e_new, u_new)


# speedup vs baseline: 1.5879x; 1.3480x over previous
import functools

import jax
import jax.numpy as jnp
import numpy as np
from jax import lax
from jax.experimental import pallas as pl
from jax.experimental.pallas import tpu as pltpu
from jax.experimental.pallas import tpu_sc as plsc

N = 10000
E = 320000
D = 128
B = 8

NC = 2
NS = 16
NW = NC * NS
CH = 128

EB = 1280
NEB = E // EB

TRB = 624
TEX = N - NS * TRB
ZR = 104

RB = 1000
NRB = N // RB
RB1 = 2000
NRB1 = N // RB1


def _iota_oh(b):
    return (b[:, None] == lax.broadcasted_iota(jnp.int32, (1, B), 1)).astype(
        jnp.float32)


def _pre_body(x_ref, b3_ref, u_ref, wa_ref, wb_ref, wd_ref, be1_ref,
              p1_ref, p2_ref):
    x = x_ref[...]
    oh = _iota_oh(b3_ref[0, 0, :])
    u1 = jnp.dot(u_ref[...], wd_ref[...]) + be1_ref[...]
    p1_ref[...] = jnp.dot(x, wa_ref[...]) + jnp.dot(oh, u1)
    p2_ref[...] = jnp.dot(x, wb_ref[...])


def _precompute(x, batch3, u, wa, wb, wd, be1):
    return pl.pallas_call(
        _pre_body,
        grid=(NRB1,),
        in_specs=[
            pl.BlockSpec((RB1, D), lambda i: (i, 0)),
            pl.BlockSpec((1, 1, RB1), lambda i: (i, 0, 0)),
            pl.BlockSpec((B, D), lambda i: (0, 0)),
            pl.BlockSpec((D, D), lambda i: (0, 0)),
            pl.BlockSpec((D, D), lambda i: (0, 0)),
            pl.BlockSpec((D, D), lambda i: (0, 0)),
            pl.BlockSpec((D,), lambda i: (0,)),
        ],
        out_specs=[
            pl.BlockSpec((RB1, D), lambda i: (i, 0)),
            pl.BlockSpec((RB1, D), lambda i: (i, 0)),
        ],
        out_shape=[
            jax.ShapeDtypeStruct((N, D), jnp.float32),
            jax.ShapeDtypeStruct((N, D), jnp.float32),
        ],
    )(x, batch3, u, wa, wb, wd, be1)


def _sc_gather(p1, p2, srch, dsth, batchv, eh):
    ew = eh // NW
    nfull = ew // CH
    tail = ew - nfull * CH
    mesh = plsc.VectorSubcoreMesh(core_axis_name="c", subcore_axis_name="s",
                                  num_cores=NC, num_subcores=NS)

    @functools.partial(
        pl.kernel,
        out_type=(
            jax.ShapeDtypeStruct((eh, D), jnp.float32),
            jax.ShapeDtypeStruct((eh, D), jnp.float32),
            jax.ShapeDtypeStruct((eh,), jnp.int32),
        ),
        mesh=mesh,
        compiler_params=pltpu.CompilerParams(needs_layout_passes=False),
        scratch_types=[
            pltpu.VMEM((ew,), jnp.int32),
            pltpu.VMEM((ew,), jnp.int32),
            pltpu.VMEM((ew,), jnp.int32),
            pltpu.VMEM((N,), jnp.int32),
            pltpu.VMEM((2, CH, D), jnp.float32),
            pltpu.VMEM((2, CH, D), jnp.float32),
            pltpu.VMEM((max(tail, 8), D), jnp.float32),
            pltpu.VMEM((max(tail, 8), D), jnp.float32),
            pltpu.SemaphoreType.DMA,
            pltpu.SemaphoreType.DMA,
        ],
    )
    def k(p1_hbm, p2_hbm, src_hbm, dst_hbm, batch_hbm,
          gs_hbm, gd_hbm, bsrc_hbm,
          sidx_all, didx_all, bsrc_all, batch_v,
          rows_a, rows_b, rows_at, rows_bt, gsem, wsem):
        wid = lax.axis_index("s") * NC + lax.axis_index("c")
        wbase = wid * ew
        pltpu.sync_copy(src_hbm.at[pl.ds(wbase, ew)], sidx_all)
        pltpu.sync_copy(dst_hbm.at[pl.ds(wbase, ew)], didx_all)
        pltpu.sync_copy(batch_hbm, batch_v)

        def fire(j, buf):
            pltpu.async_copy(
                p1_hbm.at[sidx_all.at[pl.ds(j * CH, CH)]], rows_a.at[buf],
                gsem)
            pltpu.async_copy(
                p2_hbm.at[didx_all.at[pl.ds(j * CH, CH)]], rows_b.at[buf],
                gsem)

        fire(0, 0)

        def body(j, carry):
            p = j & 1
            q = 1 - p
            pltpu.make_async_copy(p1_hbm.at[sidx_all.at[pl.ds(0, CH)]],
                                  rows_a.at[p], gsem).wait()
            pltpu.make_async_copy(p1_hbm.at[sidx_all.at[pl.ds(0, CH)]],
                                  rows_b.at[p], gsem).wait()

            @pl.when(j >= 1)
            def _():
                pltpu.make_async_copy(rows_a.at[q],
                                      gs_hbm.at[pl.ds(0, CH)], wsem).wait()
                pltpu.make_async_copy(rows_b.at[q],
                                      gd_hbm.at[pl.ds(0, CH)], wsem).wait()

            @pl.when(j < nfull - 1)
            def _():
                fire(j + 1, q)

            for kk in range(CH // 16):
                off = pl.ds(j * CH + kk * 16, 16)
                bsrc_all[off] = plsc.load_gather(batch_v, [sidx_all[off]])
            pltpu.async_copy(rows_a.at[p],
                             gs_hbm.at[pl.ds(wbase + j * CH, CH)], wsem)
            pltpu.async_copy(rows_b.at[p],
                             gd_hbm.at[pl.ds(wbase + j * CH, CH)], wsem)
            return carry

        lax.fori_loop(0, nfull, body, 0)
        lastb = (nfull - 1) & 1
        pltpu.make_async_copy(rows_a.at[lastb],
                              gs_hbm.at[pl.ds(0, CH)], wsem).wait()
        pltpu.make_async_copy(rows_b.at[lastb],
                              gd_hbm.at[pl.ds(0, CH)], wsem).wait()

        if tail:
            tb = nfull * CH
            c1 = pltpu.async_copy(p1_hbm.at[sidx_all.at[pl.ds(tb, tail)]],
                                  rows_at, gsem)
            c2 = pltpu.async_copy(p2_hbm.at[didx_all.at[pl.ds(tb, tail)]],
                                  rows_bt, gsem)
            c1.wait()
            c2.wait()
            off = pl.ds(tb, tail)
            bsrc_all[off] = plsc.load_gather(batch_v, [sidx_all[off]])
            pltpu.sync_copy(rows_at, gs_hbm.at[pl.ds(wbase + tb, tail)])
            pltpu.sync_copy(rows_bt, gd_hbm.at[pl.ds(wbase + tb, tail)])
        pltpu.sync_copy(bsrc_all, bsrc_hbm.at[pl.ds(wbase, ew)])

    return k(p1, p2, srch, dsth, batchv)


def _edge_body(nblk, ea_ref, gs_ref, gd_ref, b3_ref, wc_ref, w2_ref, be2_ref,
               en_ref, eg_ref, acc_ref):
    i = pl.program_id(0)
    ea = ea_ref[...]
    h = jnp.maximum(gs_ref[...] + gd_ref[...] + jnp.dot(ea, wc_ref[...]), 0.0)
    en = ea + jnp.dot(h, w2_ref[...]) + be2_ref[...]
    en_ref[...] = en
    oh = _iota_oh(b3_ref[0, 0, :])

    @pl.when(i == 0)
    def _():
        acc_ref[...] = jnp.zeros_like(acc_ref)

    acc_ref[...] += lax.dot_general(oh, en, (((0,), (0,)), ((), ())))

    @pl.when(i == nblk - 1)
    def _():
        eg_ref[...] = acc_ref[...]


def _edge_mlp_half(edge_attr, gs, gd, bsrc3, wc, w2, be2, off_b, nblk,
                   en_prev=None):
    ebs = pl.BlockSpec((EB, D), lambda i: (i + off_b, 0))
    kwargs = {}
    ins = [edge_attr, gs, gd, bsrc3, wc, w2, be2]
    in_specs = [
        ebs,
        pl.BlockSpec((EB, D), lambda i: (i, 0)),
        pl.BlockSpec((EB, D), lambda i: (i, 0)),
        pl.BlockSpec((1, 1, EB), lambda i: (i, 0, 0)),
        pl.BlockSpec((D, D), lambda i: (0, 0)),
        pl.BlockSpec((D, D), lambda i: (0, 0)),
        pl.BlockSpec((D,), lambda i: (0,)),
    ]
    if en_prev is not None:
        ins.append(en_prev)
        in_specs.append(pl.BlockSpec(memory_space=pl.ANY))
        kwargs["input_output_aliases"] = {7: 0}

    def body(*refs):
        if en_prev is not None:
            refs = refs[:7] + refs[8:]
        _edge_body(nblk, *refs)

    return pl.pallas_call(
        body,
        grid=(nblk,),
        in_specs=in_specs,
        out_specs=[
            ebs,
            pl.BlockSpec((B, D), lambda i: (0, 0)),
        ],
        out_shape=[
            jax.ShapeDtypeStruct((E, D), jnp.float32),
            jax.ShapeDtypeStruct((B, D), jnp.float32),
        ],
        scratch_shapes=[pltpu.VMEM((B, D), jnp.float32)],
        **kwargs,
    )(*ins)


def _sc_scatter(e_new, dsth, eoff, eh):
    ew = eh // NW
    nfull = ew // CH
    tail = ew - nfull * CH
    mesh = plsc.VectorSubcoreMesh(core_axis_name="c", subcore_axis_name="s",
                                  num_cores=NC, num_subcores=NS)

    @functools.partial(
        pl.kernel,
        out_type=jax.ShapeDtypeStruct((NC * N, D), jnp.float32),
        mesh=mesh,
        compiler_params=pltpu.CompilerParams(needs_layout_passes=False),
        scratch_types=[
            pltpu.VMEM_SHARED((N, D), jnp.float32),
            pltpu.VMEM((2, CH), jnp.int32),
            pltpu.VMEM((2, CH, D), jnp.float32),
            pltpu.VMEM((max(tail, 8),), jnp.int32),
            pltpu.VMEM((max(tail, 8), D), jnp.float32),
            pltpu.VMEM((ZR, D), jnp.float32),
            pltpu.SemaphoreType.DMA,
            pltpu.SemaphoreType.DMA,
        ],
    )
    def k(en_hbm, dst_hbm, out_hbm,
          spmem, didx, rows, didx_t, rows_t, zbuf, isem, rsem):
        cid = lax.axis_index("c")
        sid = lax.axis_index("s")
        wid = sid * NC + cid

        def zb(r, carry):
            for kk in range(D // 16):
                zbuf[r, pl.ds(kk * 16, 16)] = jnp.zeros((16,), jnp.float32)
            return carry

        lax.fori_loop(0, ZR, zb, 0)
        tbase = sid * TRB
        for kk in range(TRB // ZR):
            pltpu.sync_copy(zbuf, spmem.at[pl.ds(tbase + kk * ZR, ZR)])

        @pl.when(sid == 0)
        def _():
            pltpu.sync_copy(zbuf.at[pl.ds(0, TEX)],
                            spmem.at[pl.ds(NS * TRB, TEX)])

        plsc.subcore_barrier()

        def fire(j, buf):
            base = wid * ew + j * CH
            pltpu.async_copy(dst_hbm.at[pl.ds(base, CH)], didx.at[buf], isem)
            pltpu.async_copy(en_hbm.at[pl.ds(eoff + base, CH)],
                             rows.at[buf], rsem)

        fire(0, 0)

        def body(j, carry):
            p = j & 1
            q = 1 - p
            pltpu.make_async_copy(dst_hbm.at[pl.ds(0, CH)],
                                  didx.at[p], isem).wait()
            pltpu.make_async_copy(en_hbm.at[pl.ds(0, CH)],
                                  rows.at[p], rsem).wait()

            @pl.when(j < nfull - 1)
            def _():
                fire(j + 1, q)

            pltpu.sync_copy(rows.at[p], spmem.at[didx.at[p]], add=True)
            return carry

        lax.fori_loop(0, nfull, body, 0)
        if tail:
            tb = wid * ew + nfull * CH
            pltpu.sync_copy(dst_hbm.at[pl.ds(tb, tail)], didx_t)
            pltpu.sync_copy(en_hbm.at[pl.ds(eoff + tb, tail)], rows_t)
            pltpu.sync_copy(rows_t, spmem.at[didx_t], add=True)
        plsc.subcore_barrier()
        pltpu.sync_copy(spmem.at[pl.ds(tbase, TRB)],
                        out_hbm.at[pl.ds(cid * N + tbase, TRB)])

        @pl.when(sid == 0)
        def _():
            pltpu.sync_copy(spmem.at[pl.ds(NS * TRB, TEX)],
                            out_hbm.at[pl.ds(cid * N + NS * TRB, TEX)])

    return k(e_new, dsth)


def _node_body(x_ref, a0_ref, a1_ref, b3_ref, u_ref, eg_ref,
               wna_ref, wnb_ref, wnc_ref, bn1_ref, wn2_ref, bn2_ref,
               wga_ref, wgb_ref, wgc_ref, bg1_ref, wg2_ref, bg2_ref,
               xn_ref, un_ref, acc_ref):
    i = pl.program_id(0)
    x = x_ref[...]
    agg = a0_ref[...] + a1_ref[...]
    oh = _iota_oh(b3_ref[0, 0, :])
    u = u_ref[...]
    u3 = jnp.dot(u, wnc_ref[...])
    h = jnp.maximum(
        jnp.dot(x, wna_ref[...]) + jnp.dot(agg, wnb_ref[...])
        + jnp.dot(oh, u3) + bn1_ref[...], 0.0)
    xn = x + jnp.dot(h, wn2_ref[...]) + bn2_ref[...]
    xn_ref[...] = xn

    @pl.when(i == 0)
    def _():
        acc_ref[...] = jnp.zeros_like(acc_ref)

    acc_ref[...] += lax.dot_general(oh, xn, (((0,), (0,)), ((), ())))

    @pl.when(i == NRB - 1)
    def _():
        ng = acc_ref[...]
        eg = eg_ref[...]
        g = jnp.maximum(
            jnp.dot(ng, wga_ref[...]) + jnp.dot(eg, wgb_ref[...])
            + jnp.dot(u, wgc_ref[...]) + bg1_ref[...], 0.0)
        un_ref[...] = u + jnp.dot(g, wg2_ref[...]) + bg2_ref[...]


def _node_global(x, aggp, batch3, u, eg,
                 wna, wnb, wnc, bn1, wn2, bn2,
                 wga, wgb, wgc, bg1, wg2, bg2):
    wspec = pl.BlockSpec((D, D), lambda i: (0, 0))
    bspec = pl.BlockSpec((D,), lambda i: (0,))
    uspec = pl.BlockSpec((B, D), lambda i: (0, 0))
    return pl.pallas_call(
        _node_body,
        grid=(NRB,),
        in_specs=[
            pl.BlockSpec((RB, D), lambda i: (i, 0)),
            pl.BlockSpec((RB, D), lambda i: (i, 0)),
            pl.BlockSpec((RB, D), lambda i: (i + NRB, 0)),
            pl.BlockSpec((1, 1, RB), lambda i: (i, 0, 0)),
            uspec, uspec,
            wspec, wspec, wspec, bspec, wspec, bspec,
            wspec, wspec, wspec, bspec, wspec, bspec,
        ],
        out_specs=[
            pl.BlockSpec((RB, D), lambda i: (i, 0)),
            pl.BlockSpec((B, D), lambda i: (0, 0)),
        ],
        out_shape=[
            jax.ShapeDtypeStruct((N, D), jnp.float32),
            jax.ShapeDtypeStruct((B, D), jnp.float32),
        ],
        scratch_shapes=[pltpu.VMEM((B, D), jnp.float32)],
    )(x, aggp, aggp, batch3, u, eg,
      wna, wnb, wnc, bn1, wn2, bn2,
      wga, wgb, wgc, bg1, wg2, bg2)


def kernel(x, edge_attr, u, edge_index, batch,
           We1, be1, We2, be2,
           Wn1, bn1, Wn2, bn2,
           Wg1, bg1, Wg2, bg2):
    src = edge_index[0].astype(jnp.int32)
    dst = edge_index[1].astype(jnp.int32)
    batch32 = batch.astype(jnp.int32)
    batch3 = batch32.reshape(NRB, 1, RB)

    p1, p2 = _precompute(x, batch32.reshape(NRB1, 1, RB1), u,
                         We1[:D], We1[D:2 * D], We1[3 * D:], be1)

    gs, gd, bsrc = _sc_gather(p1, p2, src, dst, batch32, E)
    e_new, eg = _edge_mlp_half(edge_attr, gs, gd,
                               bsrc.reshape(NEB, 1, EB),
                               We1[2 * D:3 * D], We2, be2, 0, NEB)
    aggp = _sc_scatter(e_new, dst, 0, E)

    x_new, u_new = _node_global(
        x, aggp, batch3, u, eg,
        Wn1[:D], Wn1[D:2 * D], Wn1[2 * D:], bn1, Wn2, bn2,
        Wg1[:D], Wg1[D:2 * D], Wg1[2 * D:], bg1, Wg2, bg2)
    return (x_new, e_new, u_new)


# speedup vs baseline: 1.8306x; 1.1528x over previous
import functools

import jax
import jax.numpy as jnp
from jax import lax
from jax.experimental import pallas as pl
from jax.experimental.pallas import tpu as pltpu
from jax.experimental.pallas import tpu_sc as plsc

N = 10000
E = 320000
D = 128
B = 8

NC = 2
NS = 16
NW = NC * NS
CH = 128

EB = 2560
NEB = E // EB

TRB = 624
TEX = N - NS * TRB
ZR = 104

RB = 2000
NRB = N // RB
RB1 = 2000
NRB1 = N // RB1


def _iota_oh(b):
    return (b[:, None] == lax.broadcasted_iota(jnp.int32, (1, B), 1)).astype(
        jnp.float32)


def _pre_body(x_ref, b3_ref, u_ref, wa_ref, wb_ref, wd_ref, be1_ref,
              p1_ref, p2_ref):
    x = x_ref[...]
    oh = _iota_oh(b3_ref[0, 0, :])
    u1 = jnp.dot(u_ref[...], wd_ref[...]) + be1_ref[...]
    p1_ref[...] = jnp.dot(x, wa_ref[...]) + jnp.dot(oh, u1)
    p2_ref[...] = jnp.dot(x, wb_ref[...])


def _precompute(x, batch3, u, wa, wb, wd, be1):
    return pl.pallas_call(
        _pre_body,
        grid=(NRB1,),
        in_specs=[
            pl.BlockSpec((RB1, D), lambda i: (i, 0)),
            pl.BlockSpec((1, 1, RB1), lambda i: (i, 0, 0)),
            pl.BlockSpec((B, D), lambda i: (0, 0)),
            pl.BlockSpec((D, D), lambda i: (0, 0)),
            pl.BlockSpec((D, D), lambda i: (0, 0)),
            pl.BlockSpec((D, D), lambda i: (0, 0)),
            pl.BlockSpec((D,), lambda i: (0,)),
        ],
        out_specs=[
            pl.BlockSpec((RB1, D), lambda i: (i, 0)),
            pl.BlockSpec((RB1, D), lambda i: (i, 0)),
        ],
        out_shape=[
            jax.ShapeDtypeStruct((N, D), jnp.float32),
            jax.ShapeDtypeStruct((N, D), jnp.float32),
        ],
    )(x, batch3, u, wa, wb, wd, be1)


def _sc_gather(p1, p2, srch, dsth, batchv, eh):
    ew = eh // NW
    nfull = ew // CH
    tail = ew - nfull * CH
    mesh = plsc.VectorSubcoreMesh(core_axis_name="c", subcore_axis_name="s",
                                  num_cores=NC, num_subcores=NS)

    @functools.partial(
        pl.kernel,
        out_type=(
            jax.ShapeDtypeStruct((eh, D), jnp.float32),
            jax.ShapeDtypeStruct((eh, D), jnp.float32),
            jax.ShapeDtypeStruct((eh,), jnp.int32),
        ),
        mesh=mesh,
        compiler_params=pltpu.CompilerParams(needs_layout_passes=False),
        scratch_types=[
            pltpu.VMEM((ew,), jnp.int32),
            pltpu.VMEM((ew,), jnp.int32),
            pltpu.VMEM((ew,), jnp.int32),
            pltpu.VMEM((N,), jnp.int32),
            pltpu.VMEM((2, CH, D), jnp.float32),
            pltpu.VMEM((2, CH, D), jnp.float32),
            pltpu.VMEM((max(tail, 8), D), jnp.float32),
            pltpu.VMEM((max(tail, 8), D), jnp.float32),
            pltpu.SemaphoreType.DMA,
            pltpu.SemaphoreType.DMA,
        ],
    )
    def k(p1_hbm, p2_hbm, src_hbm, dst_hbm, batch_hbm,
          gs_hbm, gd_hbm, bsrc_hbm,
          sidx_all, didx_all, bsrc_all, batch_v,
          rows_a, rows_b, rows_at, rows_bt, gsem, wsem):
        wid = lax.axis_index("s") * NC + lax.axis_index("c")
        wbase = wid * ew
        pltpu.sync_copy(src_hbm.at[pl.ds(wbase, ew)], sidx_all)
        pltpu.sync_copy(dst_hbm.at[pl.ds(wbase, ew)], didx_all)
        pltpu.sync_copy(batch_hbm, batch_v)

        def fire(j, buf):
            pltpu.async_copy(
                p1_hbm.at[sidx_all.at[pl.ds(j * CH, CH)]], rows_a.at[buf],
                gsem)
            pltpu.async_copy(
                p2_hbm.at[didx_all.at[pl.ds(j * CH, CH)]], rows_b.at[buf],
                gsem)

        fire(0, 0)

        def body(j, carry):
            p = j & 1
            q = 1 - p
            pltpu.make_async_copy(p1_hbm.at[sidx_all.at[pl.ds(0, CH)]],
                                  rows_a.at[p], gsem).wait()
            pltpu.make_async_copy(p1_hbm.at[sidx_all.at[pl.ds(0, CH)]],
                                  rows_b.at[p], gsem).wait()

            @pl.when(j >= 1)
            def _():
                pltpu.make_async_copy(rows_a.at[q],
                                      gs_hbm.at[pl.ds(0, CH)], wsem).wait()
                pltpu.make_async_copy(rows_b.at[q],
                                      gd_hbm.at[pl.ds(0, CH)], wsem).wait()

            @pl.when(j < nfull - 1)
            def _():
                fire(j + 1, q)

            for kk in range(CH // 16):
                off = pl.ds(j * CH + kk * 16, 16)
                bsrc_all[off] = plsc.load_gather(batch_v, [sidx_all[off]])
            pltpu.async_copy(rows_a.at[p],
                             gs_hbm.at[pl.ds(wbase + j * CH, CH)], wsem)
            pltpu.async_copy(rows_b.at[p],
                             gd_hbm.at[pl.ds(wbase + j * CH, CH)], wsem)
            return carry

        lax.fori_loop(0, nfull, body, 0)
        lastb = (nfull - 1) & 1
        pltpu.make_async_copy(rows_a.at[lastb],
                              gs_hbm.at[pl.ds(0, CH)], wsem).wait()
        pltpu.make_async_copy(rows_b.at[lastb],
                              gd_hbm.at[pl.ds(0, CH)], wsem).wait()

        if tail:
            tb = nfull * CH
            c1 = pltpu.async_copy(p1_hbm.at[sidx_all.at[pl.ds(tb, tail)]],
                                  rows_at, gsem)
            c2 = pltpu.async_copy(p2_hbm.at[didx_all.at[pl.ds(tb, tail)]],
                                  rows_bt, gsem)
            c1.wait()
            c2.wait()
            off = pl.ds(tb, tail)
            bsrc_all[off] = plsc.load_gather(batch_v, [sidx_all[off]])
            pltpu.sync_copy(rows_at, gs_hbm.at[pl.ds(wbase + tb, tail)])
            pltpu.sync_copy(rows_bt, gd_hbm.at[pl.ds(wbase + tb, tail)])
        pltpu.sync_copy(bsrc_all, bsrc_hbm.at[pl.ds(wbase, ew)])

    return k(p1, p2, srch, dsth, batchv)


def _edge_body(nblk, ea_ref, gs_ref, gd_ref, b3_ref, wc_ref, w2_ref, be2_ref,
               en_ref, eg_ref, acc_ref):
    i = pl.program_id(0)
    ea = ea_ref[...]
    h = jnp.maximum(gs_ref[...] + gd_ref[...] + jnp.dot(ea, wc_ref[...]), 0.0)
    en = ea + jnp.dot(h, w2_ref[...]) + be2_ref[...]
    en_ref[...] = en
    oh = _iota_oh(b3_ref[0, 0, :])

    @pl.when(i == 0)
    def _():
        acc_ref[...] = jnp.zeros_like(acc_ref)

    acc_ref[...] += lax.dot_general(oh, en, (((0,), (0,)), ((), ())))

    @pl.when(i == nblk - 1)
    def _():
        eg_ref[...] = acc_ref[...]


def _edge_mlp_half(edge_attr, gs, gd, bsrc3, wc, w2, be2, off_b, nblk,
                   en_prev=None):
    ebs = pl.BlockSpec((EB, D), lambda i: (i + off_b, 0))
    kwargs = {}
    ins = [edge_attr, gs, gd, bsrc3, wc, w2, be2]
    in_specs = [
        ebs,
        pl.BlockSpec((EB, D), lambda i: (i, 0)),
        pl.BlockSpec((EB, D), lambda i: (i, 0)),
        pl.BlockSpec((1, 1, EB), lambda i: (i, 0, 0)),
        pl.BlockSpec((D, D), lambda i: (0, 0)),
        pl.BlockSpec((D, D), lambda i: (0, 0)),
        pl.BlockSpec((D,), lambda i: (0,)),
    ]
    if en_prev is not None:
        ins.append(en_prev)
        in_specs.append(pl.BlockSpec(memory_space=pl.ANY))
        kwargs["input_output_aliases"] = {7: 0}

    def body(*refs):
        if en_prev is not None:
            refs = refs[:7] + refs[8:]
        _edge_body(nblk, *refs)

    return pl.pallas_call(
        body,
        grid=(nblk,),
        in_specs=in_specs,
        out_specs=[
            ebs,
            pl.BlockSpec((B, D), lambda i: (0, 0)),
        ],
        out_shape=[
            jax.ShapeDtypeStruct((E, D), jnp.float32),
            jax.ShapeDtypeStruct((B, D), jnp.float32),
        ],
        scratch_shapes=[pltpu.VMEM((B, D), jnp.float32)],
        **kwargs,
    )(*ins)


def _sc_scatter(e_new, dsth, eoff, eh):
    ew = eh // NW
    nfull = ew // CH
    tail = ew - nfull * CH
    mesh = plsc.VectorSubcoreMesh(core_axis_name="c", subcore_axis_name="s",
                                  num_cores=NC, num_subcores=NS)

    @functools.partial(
        pl.kernel,
        out_type=jax.ShapeDtypeStruct((NC * N, D), jnp.float32),
        mesh=mesh,
        compiler_params=pltpu.CompilerParams(needs_layout_passes=False),
        scratch_types=[
            pltpu.VMEM_SHARED((N, D), jnp.float32),
            pltpu.VMEM((2, CH), jnp.int32),
            pltpu.VMEM((2, CH, D), jnp.float32),
            pltpu.VMEM((max(tail, 8),), jnp.int32),
            pltpu.VMEM((max(tail, 8), D), jnp.float32),
            pltpu.VMEM((ZR, D), jnp.float32),
            pltpu.SemaphoreType.DMA,
            pltpu.SemaphoreType.DMA,
        ],
    )
    def k(en_hbm, dst_hbm, out_hbm,
          spmem, didx, rows, didx_t, rows_t, zbuf, isem, rsem):
        cid = lax.axis_index("c")
        sid = lax.axis_index("s")
        wid = sid * NC + cid

        def zb(r, carry):
            for kk in range(D // 16):
                zbuf[r, pl.ds(kk * 16, 16)] = jnp.zeros((16,), jnp.float32)
            return carry

        lax.fori_loop(0, ZR, zb, 0)
        tbase = sid * TRB
        for kk in range(TRB // ZR):
            pltpu.sync_copy(zbuf, spmem.at[pl.ds(tbase + kk * ZR, ZR)])

        @pl.when(sid == 0)
        def _():
            pltpu.sync_copy(zbuf.at[pl.ds(0, TEX)],
                            spmem.at[pl.ds(NS * TRB, TEX)])

        plsc.subcore_barrier()

        def fire(j, buf):
            base = wid * ew + j * CH
            pltpu.async_copy(dst_hbm.at[pl.ds(base, CH)], didx.at[buf], isem)
            pltpu.async_copy(en_hbm.at[pl.ds(eoff + base, CH)],
                             rows.at[buf], rsem)

        fire(0, 0)

        def body(j, carry):
            p = j & 1
            q = 1 - p
            pltpu.make_async_copy(dst_hbm.at[pl.ds(0, CH)],
                                  didx.at[p], isem).wait()
            pltpu.make_async_copy(en_hbm.at[pl.ds(0, CH)],
                                  rows.at[p], rsem).wait()

            @pl.when(j < nfull - 1)
            def _():
                fire(j + 1, q)

            pltpu.sync_copy(rows.at[p], spmem.at[didx.at[p]], add=True)
            return carry

        lax.fori_loop(0, nfull, body, 0)
        if tail:
            tb = wid * ew + nfull * CH
            pltpu.sync_copy(dst_hbm.at[pl.ds(tb, tail)], didx_t)
            pltpu.sync_copy(en_hbm.at[pl.ds(eoff + tb, tail)], rows_t)
            pltpu.sync_copy(rows_t, spmem.at[didx_t], add=True)
        plsc.subcore_barrier()
        pltpu.sync_copy(spmem.at[pl.ds(tbase, TRB)],
                        out_hbm.at[pl.ds(cid * N + tbase, TRB)])

        @pl.when(sid == 0)
        def _():
            pltpu.sync_copy(spmem.at[pl.ds(NS * TRB, TEX)],
                            out_hbm.at[pl.ds(cid * N + NS * TRB, TEX)])

    return k(e_new, dsth)


def _node_body(x_ref, a0_ref, a1_ref, b3_ref, u_ref, eg_ref,
               wna_ref, wnb_ref, wnc_ref, bn1_ref, wn2_ref, bn2_ref,
               wga_ref, wgb_ref, wgc_ref, bg1_ref, wg2_ref, bg2_ref,
               xn_ref, un_ref, acc_ref):
    i = pl.program_id(0)
    x = x_ref[...]
    agg = a0_ref[...] + a1_ref[...]
    oh = _iota_oh(b3_ref[0, 0, :])
    u = u_ref[...]
    u3 = jnp.dot(u, wnc_ref[...])
    h = jnp.maximum(
        jnp.dot(x, wna_ref[...]) + jnp.dot(agg, wnb_ref[...])
        + jnp.dot(oh, u3) + bn1_ref[...], 0.0)
    xn = x + jnp.dot(h, wn2_ref[...]) + bn2_ref[...]
    xn_ref[...] = xn

    @pl.when(i == 0)
    def _():
        acc_ref[...] = jnp.zeros_like(acc_ref)

    acc_ref[...] += lax.dot_general(oh, xn, (((0,), (0,)), ((), ())))

    @pl.when(i == NRB - 1)
    def _():
        ng = acc_ref[...]
        eg = eg_ref[...]
        g = jnp.maximum(
            jnp.dot(ng, wga_ref[...]) + jnp.dot(eg, wgb_ref[...])
            + jnp.dot(u, wgc_ref[...]) + bg1_ref[...], 0.0)
        un_ref[...] = u + jnp.dot(g, wg2_ref[...]) + bg2_ref[...]


def _node_global(x, aggp, batch3, u, eg,
                 wna, wnb, wnc, bn1, wn2, bn2,
                 wga, wgb, wgc, bg1, wg2, bg2):
    wspec = pl.BlockSpec((D, D), lambda i: (0, 0))
    bspec = pl.BlockSpec((D,), lambda i: (0,))
    uspec = pl.BlockSpec((B, D), lambda i: (0, 0))
    return pl.pallas_call(
        _node_body,
        grid=(NRB,),
        in_specs=[
            pl.BlockSpec((RB, D), lambda i: (i, 0)),
            pl.BlockSpec((RB, D), lambda i: (i, 0)),
            pl.BlockSpec((RB, D), lambda i: (i + NRB, 0)),
            pl.BlockSpec((1, 1, RB), lambda i: (i, 0, 0)),
            uspec, uspec,
            wspec, wspec, wspec, bspec, wspec, bspec,
            wspec, wspec, wspec, bspec, wspec, bspec,
        ],
        out_specs=[
            pl.BlockSpec((RB, D), lambda i: (i, 0)),
            pl.BlockSpec((B, D), lambda i: (0, 0)),
        ],
        out_shape=[
            jax.ShapeDtypeStruct((N, D), jnp.float32),
            jax.ShapeDtypeStruct((B, D), jnp.float32),
        ],
        scratch_shapes=[pltpu.VMEM((B, D), jnp.float32)],
    )(x, aggp, aggp, batch3, u, eg,
      wna, wnb, wnc, bn1, wn2, bn2,
      wga, wgb, wgc, bg1, wg2, bg2)


def kernel(x, edge_attr, u, edge_index, batch,
           We1, be1, We2, be2,
           Wn1, bn1, Wn2, bn2,
           Wg1, bg1, Wg2, bg2):
    src = edge_index[0].astype(jnp.int32)
    dst = edge_index[1].astype(jnp.int32)
    batch32 = batch.astype(jnp.int32)
    batch3 = batch32.reshape(NRB, 1, RB)

    p1, p2 = _precompute(x, batch32.reshape(NRB1, 1, RB1), u,
                         We1[:D], We1[D:2 * D], We1[3 * D:], be1)

    gs, gd, bsrc = _sc_gather(p1, p2, src, dst, batch32, E)
    e_new, eg = _edge_mlp_half(edge_attr, gs, gd,
                               bsrc.reshape(NEB, 1, EB),
                               We1[2 * D:3 * D], We2, be2, 0, NEB)
    aggp = _sc_scatter(e_new, dst, 0, E)

    x_new, u_new = _node_global(
        x, aggp, batch3, u, eg,
        Wn1[:D], Wn1[D:2 * D], Wn1[2 * D:], bn1, Wn2, bn2,
        Wg1[:D], Wg1[D:2 * D], Wg1[2 * D:], bg1, Wg2, bg2)
    return (x_new, e_new, u_new)


# speedup vs baseline: 1.9418x; 1.0608x over previous
import functools

import jax
import jax.numpy as jnp
from jax import lax
from jax.experimental import pallas as pl
from jax.experimental.pallas import tpu as pltpu
from jax.experimental.pallas import tpu_sc as plsc

N = 10000
E = 320000
D = 128
B = 8

NC = 2
NS = 16
NW = NC * NS
CH = 128

EB = 6400
NEB = E // EB

TRB = 624
TEX = N - NS * TRB
ZR = 104

RB = 2000
NRB = N // RB
RB1 = 2000
NRB1 = N // RB1


def _iota_oh(b):
    return (b[:, None] == lax.broadcasted_iota(jnp.int32, (1, B), 1)).astype(
        jnp.float32)


def _pre_body(x_ref, b3_ref, u_ref, wa_ref, wb_ref, wd_ref, be1_ref,
              p1_ref, p2_ref):
    x = x_ref[...]
    oh = _iota_oh(b3_ref[0, 0, :])
    u1 = jnp.dot(u_ref[...], wd_ref[...]) + be1_ref[...]
    p1_ref[...] = jnp.dot(x, wa_ref[...]) + jnp.dot(oh, u1)
    p2_ref[...] = jnp.dot(x, wb_ref[...])


def _precompute(x, batch3, u, wa, wb, wd, be1):
    return pl.pallas_call(
        _pre_body,
        grid=(NRB1,),
        in_specs=[
            pl.BlockSpec((RB1, D), lambda i: (i, 0)),
            pl.BlockSpec((1, 1, RB1), lambda i: (i, 0, 0)),
            pl.BlockSpec((B, D), lambda i: (0, 0)),
            pl.BlockSpec((D, D), lambda i: (0, 0)),
            pl.BlockSpec((D, D), lambda i: (0, 0)),
            pl.BlockSpec((D, D), lambda i: (0, 0)),
            pl.BlockSpec((D,), lambda i: (0,)),
        ],
        out_specs=[
            pl.BlockSpec((RB1, D), lambda i: (i, 0)),
            pl.BlockSpec((RB1, D), lambda i: (i, 0)),
        ],
        out_shape=[
            jax.ShapeDtypeStruct((N, D), jnp.float32),
            jax.ShapeDtypeStruct((N, D), jnp.float32),
        ],
    )(x, batch3, u, wa, wb, wd, be1)


def _sc_gather(p1, p2, srch, dsth, batchv, eh):
    ew = eh // NW
    nfull = ew // CH
    tail = ew - nfull * CH
    mesh = plsc.VectorSubcoreMesh(core_axis_name="c", subcore_axis_name="s",
                                  num_cores=NC, num_subcores=NS)

    @functools.partial(
        pl.kernel,
        out_type=(
            jax.ShapeDtypeStruct((eh, D), jnp.float32),
            jax.ShapeDtypeStruct((eh, D), jnp.float32),
            jax.ShapeDtypeStruct((eh,), jnp.int32),
        ),
        mesh=mesh,
        compiler_params=pltpu.CompilerParams(needs_layout_passes=False),
        scratch_types=[
            pltpu.VMEM((ew,), jnp.int32),
            pltpu.VMEM((ew,), jnp.int32),
            pltpu.VMEM((ew,), jnp.int32),
            pltpu.VMEM((N,), jnp.int32),
            pltpu.VMEM((2, CH, D), jnp.float32),
            pltpu.VMEM((2, CH, D), jnp.float32),
            pltpu.VMEM((max(tail, 8), D), jnp.float32),
            pltpu.VMEM((max(tail, 8), D), jnp.float32),
            pltpu.SemaphoreType.DMA,
            pltpu.SemaphoreType.DMA,
        ],
    )
    def k(p1_hbm, p2_hbm, src_hbm, dst_hbm, batch_hbm,
          gs_hbm, gd_hbm, bsrc_hbm,
          sidx_all, didx_all, bsrc_all, batch_v,
          rows_a, rows_b, rows_at, rows_bt, gsem, wsem):
        wid = lax.axis_index("s") * NC + lax.axis_index("c")
        wbase = wid * ew
        pltpu.sync_copy(src_hbm.at[pl.ds(wbase, ew)], sidx_all)
        pltpu.sync_copy(dst_hbm.at[pl.ds(wbase, ew)], didx_all)
        pltpu.sync_copy(batch_hbm, batch_v)

        def fire(j, buf):
            pltpu.async_copy(
                p1_hbm.at[sidx_all.at[pl.ds(j * CH, CH)]], rows_a.at[buf],
                gsem)
            pltpu.async_copy(
                p2_hbm.at[didx_all.at[pl.ds(j * CH, CH)]], rows_b.at[buf],
                gsem)

        fire(0, 0)

        def body(j, carry):
            p = j & 1
            q = 1 - p
            pltpu.make_async_copy(p1_hbm.at[sidx_all.at[pl.ds(0, CH)]],
                                  rows_a.at[p], gsem).wait()
            pltpu.make_async_copy(p1_hbm.at[sidx_all.at[pl.ds(0, CH)]],
                                  rows_b.at[p], gsem).wait()

            @pl.when(j >= 1)
            def _():
                pltpu.make_async_copy(rows_a.at[q],
                                      gs_hbm.at[pl.ds(0, CH)], wsem).wait()
                pltpu.make_async_copy(rows_b.at[q],
                                      gd_hbm.at[pl.ds(0, CH)], wsem).wait()

            @pl.when(j < nfull - 1)
            def _():
                fire(j + 1, q)

            for kk in range(CH // 16):
                off = pl.ds(j * CH + kk * 16, 16)
                bsrc_all[off] = plsc.load_gather(batch_v, [sidx_all[off]])
            pltpu.async_copy(rows_a.at[p],
                             gs_hbm.at[pl.ds(wbase + j * CH, CH)], wsem)
            pltpu.async_copy(rows_b.at[p],
                             gd_hbm.at[pl.ds(wbase + j * CH, CH)], wsem)
            return carry

        lax.fori_loop(0, nfull, body, 0)
        lastb = (nfull - 1) & 1
        pltpu.make_async_copy(rows_a.at[lastb],
                              gs_hbm.at[pl.ds(0, CH)], wsem).wait()
        pltpu.make_async_copy(rows_b.at[lastb],
                              gd_hbm.at[pl.ds(0, CH)], wsem).wait()

        if tail:
            tb = nfull * CH
            c1 = pltpu.async_copy(p1_hbm.at[sidx_all.at[pl.ds(tb, tail)]],
                                  rows_at, gsem)
            c2 = pltpu.async_copy(p2_hbm.at[didx_all.at[pl.ds(tb, tail)]],
                                  rows_bt, gsem)
            c1.wait()
            c2.wait()
            off = pl.ds(tb, tail)
            bsrc_all[off] = plsc.load_gather(batch_v, [sidx_all[off]])
            pltpu.sync_copy(rows_at, gs_hbm.at[pl.ds(wbase + tb, tail)])
            pltpu.sync_copy(rows_bt, gd_hbm.at[pl.ds(wbase + tb, tail)])
        pltpu.sync_copy(bsrc_all, bsrc_hbm.at[pl.ds(wbase, ew)])

    return k(p1, p2, srch, dsth, batchv)


def _edge_body(nblk, ea_ref, gs_ref, gd_ref, b3_ref, wc_ref, w2_ref, be2_ref,
               en_ref, eg_ref, acc_ref):
    i = pl.program_id(0)
    ea = ea_ref[...]
    h = jnp.maximum(gs_ref[...] + gd_ref[...] + jnp.dot(ea, wc_ref[...]), 0.0)
    en = ea + jnp.dot(h, w2_ref[...]) + be2_ref[...]
    en_ref[...] = en
    oh = _iota_oh(b3_ref[0, 0, :])

    @pl.when(i == 0)
    def _():
        acc_ref[...] = jnp.zeros_like(acc_ref)

    acc_ref[...] += lax.dot_general(oh, en, (((0,), (0,)), ((), ())))

    @pl.when(i == nblk - 1)
    def _():
        eg_ref[...] = acc_ref[...]


def _edge_mlp_half(edge_attr, gs, gd, bsrc3, wc, w2, be2, off_b, nblk,
                   en_prev=None):
    ebs = pl.BlockSpec((EB, D), lambda i: (i + off_b, 0))
    kwargs = {}
    ins = [edge_attr, gs, gd, bsrc3, wc, w2, be2]
    in_specs = [
        ebs,
        pl.BlockSpec((EB, D), lambda i: (i, 0)),
        pl.BlockSpec((EB, D), lambda i: (i, 0)),
        pl.BlockSpec((1, 1, EB), lambda i: (i, 0, 0)),
        pl.BlockSpec((D, D), lambda i: (0, 0)),
        pl.BlockSpec((D, D), lambda i: (0, 0)),
        pl.BlockSpec((D,), lambda i: (0,)),
    ]
    if en_prev is not None:
        ins.append(en_prev)
        in_specs.append(pl.BlockSpec(memory_space=pl.ANY))
        kwargs["input_output_aliases"] = {7: 0}

    def body(*refs):
        if en_prev is not None:
            refs = refs[:7] + refs[8:]
        _edge_body(nblk, *refs)

    return pl.pallas_call(
        body,
        grid=(nblk,),
        in_specs=in_specs,
        out_specs=[
            ebs,
            pl.BlockSpec((B, D), lambda i: (0, 0)),
        ],
        out_shape=[
            jax.ShapeDtypeStruct((E, D), jnp.float32),
            jax.ShapeDtypeStruct((B, D), jnp.float32),
        ],
        scratch_shapes=[pltpu.VMEM((B, D), jnp.float32)],
        **kwargs,
    )(*ins)


def _sc_scatter(e_new, dsth, eoff, eh):
    ew = eh // NW
    nfull = ew // CH
    tail = ew - nfull * CH
    mesh = plsc.VectorSubcoreMesh(core_axis_name="c", subcore_axis_name="s",
                                  num_cores=NC, num_subcores=NS)

    @functools.partial(
        pl.kernel,
        out_type=jax.ShapeDtypeStruct((NC * N, D), jnp.float32),
        mesh=mesh,
        compiler_params=pltpu.CompilerParams(needs_layout_passes=False),
        scratch_types=[
            pltpu.VMEM_SHARED((N, D), jnp.float32),
            pltpu.VMEM((2, CH), jnp.int32),
            pltpu.VMEM((2, CH, D), jnp.float32),
            pltpu.VMEM((max(tail, 8),), jnp.int32),
            pltpu.VMEM((max(tail, 8), D), jnp.float32),
            pltpu.VMEM((ZR, D), jnp.float32),
            pltpu.SemaphoreType.DMA,
            pltpu.SemaphoreType.DMA,
        ],
    )
    def k(en_hbm, dst_hbm, out_hbm,
          spmem, didx, rows, didx_t, rows_t, zbuf, isem, rsem):
        cid = lax.axis_index("c")
        sid = lax.axis_index("s")
        wid = sid * NC + cid

        def zb(r, carry):
            for kk in range(D // 16):
                zbuf[r, pl.ds(kk * 16, 16)] = jnp.zeros((16,), jnp.float32)
            return carry

        lax.fori_loop(0, ZR, zb, 0)
        tbase = sid * TRB
        for kk in range(TRB // ZR):
            pltpu.sync_copy(zbuf, spmem.at[pl.ds(tbase + kk * ZR, ZR)])

        @pl.when(sid == 0)
        def _():
            pltpu.sync_copy(zbuf.at[pl.ds(0, TEX)],
                            spmem.at[pl.ds(NS * TRB, TEX)])

        plsc.subcore_barrier()

        def fire(j, buf):
            base = wid * ew + j * CH
            pltpu.async_copy(dst_hbm.at[pl.ds(base, CH)], didx.at[buf], isem)
            pltpu.async_copy(en_hbm.at[pl.ds(eoff + base, CH)],
                             rows.at[buf], rsem)

        fire(0, 0)

        def body(j, carry):
            p = j & 1
            q = 1 - p
            pltpu.make_async_copy(dst_hbm.at[pl.ds(0, CH)],
                                  didx.at[p], isem).wait()
            pltpu.make_async_copy(en_hbm.at[pl.ds(0, CH)],
                                  rows.at[p], rsem).wait()

            @pl.when(j < nfull - 1)
            def _():
                fire(j + 1, q)

            pltpu.sync_copy(rows.at[p], spmem.at[didx.at[p]], add=True)
            return carry

        lax.fori_loop(0, nfull, body, 0)
        if tail:
            tb = wid * ew + nfull * CH
            pltpu.sync_copy(dst_hbm.at[pl.ds(tb, tail)], didx_t)
            pltpu.sync_copy(en_hbm.at[pl.ds(eoff + tb, tail)], rows_t)
            pltpu.sync_copy(rows_t, spmem.at[didx_t], add=True)
        plsc.subcore_barrier()
        pltpu.sync_copy(spmem.at[pl.ds(tbase, TRB)],
                        out_hbm.at[pl.ds(cid * N + tbase, TRB)])

        @pl.when(sid == 0)
        def _():
            pltpu.sync_copy(spmem.at[pl.ds(NS * TRB, TEX)],
                            out_hbm.at[pl.ds(cid * N + NS * TRB, TEX)])

    return k(e_new, dsth)


def _node_body(x_ref, a0_ref, a1_ref, b3_ref, u_ref, eg_ref,
               wna_ref, wnb_ref, wnc_ref, bn1_ref, wn2_ref, bn2_ref,
               wga_ref, wgb_ref, wgc_ref, bg1_ref, wg2_ref, bg2_ref,
               xn_ref, un_ref, acc_ref):
    i = pl.program_id(0)
    x = x_ref[...]
    agg = a0_ref[...] + a1_ref[...]
    oh = _iota_oh(b3_ref[0, 0, :])
    u = u_ref[...]
    u3 = jnp.dot(u, wnc_ref[...])
    h = jnp.maximum(
        jnp.dot(x, wna_ref[...]) + jnp.dot(agg, wnb_ref[...])
        + jnp.dot(oh, u3) + bn1_ref[...], 0.0)
    xn = x + jnp.dot(h, wn2_ref[...]) + bn2_ref[...]
    xn_ref[...] = xn

    @pl.when(i == 0)
    def _():
        acc_ref[...] = jnp.zeros_like(acc_ref)

    acc_ref[...] += lax.dot_general(oh, xn, (((0,), (0,)), ((), ())))

    @pl.when(i == NRB - 1)
    def _():
        ng = acc_ref[...]
        eg = eg_ref[...]
        g = jnp.maximum(
            jnp.dot(ng, wga_ref[...]) + jnp.dot(eg, wgb_ref[...])
            + jnp.dot(u, wgc_ref[...]) + bg1_ref[...], 0.0)
        un_ref[...] = u + jnp.dot(g, wg2_ref[...]) + bg2_ref[...]


def _node_global(x, aggp, batch3, u, eg,
                 wna, wnb, wnc, bn1, wn2, bn2,
                 wga, wgb, wgc, bg1, wg2, bg2):
    wspec = pl.BlockSpec((D, D), lambda i: (0, 0))
    bspec = pl.BlockSpec((D,), lambda i: (0,))
    uspec = pl.BlockSpec((B, D), lambda i: (0, 0))
    return pl.pallas_call(
        _node_body,
        grid=(NRB,),
        in_specs=[
            pl.BlockSpec((RB, D), lambda i: (i, 0)),
            pl.BlockSpec((RB, D), lambda i: (i, 0)),
            pl.BlockSpec((RB, D), lambda i: (i + NRB, 0)),
            pl.BlockSpec((1, 1, RB), lambda i: (i, 0, 0)),
            uspec, uspec,
            wspec, wspec, wspec, bspec, wspec, bspec,
            wspec, wspec, wspec, bspec, wspec, bspec,
        ],
        out_specs=[
            pl.BlockSpec((RB, D), lambda i: (i, 0)),
            pl.BlockSpec((B, D), lambda i: (0, 0)),
        ],
        out_shape=[
            jax.ShapeDtypeStruct((N, D), jnp.float32),
            jax.ShapeDtypeStruct((B, D), jnp.float32),
        ],
        scratch_shapes=[pltpu.VMEM((B, D), jnp.float32)],
    )(x, aggp, aggp, batch3, u, eg,
      wna, wnb, wnc, bn1, wn2, bn2,
      wga, wgb, wgc, bg1, wg2, bg2)


def kernel(x, edge_attr, u, edge_index, batch,
           We1, be1, We2, be2,
           Wn1, bn1, Wn2, bn2,
           Wg1, bg1, Wg2, bg2):
    src = edge_index[0].astype(jnp.int32)
    dst = edge_index[1].astype(jnp.int32)
    batch32 = batch.astype(jnp.int32)
    batch3 = batch32.reshape(NRB, 1, RB)

    p1, p2 = _precompute(x, batch32.reshape(NRB1, 1, RB1), u,
                         We1[:D], We1[D:2 * D], We1[3 * D:], be1)

    gs, gd, bsrc = _sc_gather(p1, p2, src, dst, batch32, E)
    e_new, eg = _edge_mlp_half(edge_attr, gs, gd,
                               bsrc.reshape(NEB, 1, EB),
                               We1[2 * D:3 * D], We2, be2, 0, NEB)
    aggp = _sc_scatter(e_new, dst, 0, E)

    x_new, u_new = _node_global(
        x, aggp, batch3, u, eg,
        Wn1[:D], Wn1[D:2 * D], Wn1[2 * D:], bn1, Wn2, bn2,
        Wg1[:D], Wg1[D:2 * D], Wg1[2 * D:], bg1, Wg2, bg2)
    return (x_new, e_new, u_new)
